# Initial kernel scaffold; baseline (speedup 1.0000x reference)
import jax
import jax.numpy as jnp
from jax.experimental import pallas as pl


def kernel(x, support0_indices, support0_values, support1_indices, support1_values, fc0_W, fc0_b, fc1_W, fc1_b, conv_kernels, conv_weights):
    raise NotImplementedError("write your pallas kernel here")


# speedup vs baseline: 1.9806x; 1.9806x over previous
import functools
import math

import jax
import jax.numpy as jnp
from jax import lax
from jax.experimental import pallas as pl
from jax.experimental.pallas import tpu as pltpu
from jax.experimental.pallas import tpu_sc as plsc

N_NODE = 10000
N_FEAT = 256
N_HID = 256
N_CLASS = 64
N_LAYERS = 8
N_EDGE = 160000
LAMDA = 0.5
ALPHA = 0.1

HALF = 128
CHUNK = 128
EPAD = 163840
E_PER_TILE = EPAD // 16
N_CHUNKS = E_PER_TILE // CHUNK
ROWS_PER_TILE = 624
ROW_CHUNK = 208


def _make_spmm():
    mesh = plsc.VectorSubcoreMesh(core_axis_name="c", subcore_axis_name="s")

    scratch = [
        pltpu.VMEM((CHUNK,), jnp.int32),
        pltpu.VMEM((CHUNK,), jnp.int32),
        pltpu.VMEM((CHUNK,), jnp.float32),
        pltpu.VMEM((CHUNK, HALF), jnp.float32),
        pltpu.VMEM((ROW_CHUNK, HALF), jnp.float32),
        pltpu.VMEM_SHARED((N_NODE, HALF), jnp.float32),
        pltpu.SemaphoreType.DMA,
    ]

    def body(table, col, dst, val, out, colv, dstv, wv, rows, zbuf, acc, sem):
        c = lax.axis_index("c")
        s = lax.axis_index("s")

        def zrow(r, carry):
            for f in range(HALF // 16):
                zbuf[r, pl.ds(f * 16, 16)] = jnp.zeros((16,), jnp.float32)
            return carry
        lax.fori_loop(0, ROW_CHUNK, zrow, 0)
        for j in range(3):
            pltpu.sync_copy(
                zbuf, acc.at[pl.ds(s * ROWS_PER_TILE + j * ROW_CHUNK, ROW_CHUNK)])

        @pl.when(s == 0)
        def _():
            pltpu.sync_copy(zbuf.at[pl.ds(0, 16)], acc.at[pl.ds(9984, 16)])

        plsc.subcore_barrier()

        coff = c * N_NODE

        def chunk(g, carry):
            base = s * E_PER_TILE + g * CHUNK
            pltpu.sync_copy(col.at[pl.ds(base, CHUNK)], colv)
            pltpu.sync_copy(dst.at[pl.ds(base, CHUNK)], dstv)
            pltpu.sync_copy(val.at[pl.ds(base, CHUNK)], wv)
            for j in range(CHUNK // 16):
                colv[pl.ds(j * 16, 16)] = colv[pl.ds(j * 16, 16)] + coff
            pltpu.async_copy(table.at[colv], rows, sem).wait()
            for eg in range(CHUNK // 16):
                w16 = wv[pl.ds(eg * 16, 16)]
                for e in range(16):
                    r = eg * 16 + e
                    sp = jnp.broadcast_to(w16[e], (16,))
                    for f in range(HALF // 16):
                        rows[r, pl.ds(f * 16, 16)] = (
                            rows[r, pl.ds(f * 16, 16)] * sp)
            pltpu.sync_copy(rows, acc.at[dstv], add=True)
            return carry

        lax.fori_loop(0, N_CHUNKS, chunk, 0)

        plsc.subcore_barrier()

        for j in range(3):
            r0 = s * ROWS_PER_TILE + j * ROW_CHUNK
            pltpu.sync_copy(acc.at[pl.ds(r0, ROW_CHUNK)],
                            out.at[pl.ds(coff + r0, ROW_CHUNK)])

        @pl.when(s == 0)
        def _():
            pltpu.sync_copy(acc.at[pl.ds(9984, 16)],
                            out.at[pl.ds(coff + 9984, 16)])

    def run(*args):
        return pl.kernel(
            body,
            out_type=jax.ShapeDtypeStruct((2 * N_NODE, HALF), jnp.float32),
            mesh=mesh,
            scratch_types=scratch,
        )(*args)

    return run


_spmm = _make_spmm()


def _scale_rows(t2, k):
    def body(t_ref, k_ref, o_ref):
        o_ref[...] = t_ref[...] * k_ref[...][None, :, :]

    return pl.pallas_call(
        body,
        grid=(_NRB,),
        in_specs=[
            pl.BlockSpec((2, _RB, HALF), lambda r: (0, r, 0)),
            pl.BlockSpec((_RB, 1), lambda r: (r, 0)),
        ],
        out_specs=pl.BlockSpec((2, _RB, HALF), lambda r: (0, r, 0)),
        out_shape=jax.ShapeDtypeStruct((2, N_NODE, HALF), jnp.float32),
    )(t2, k)


_RB = 1000
_NRB = N_NODE // _RB
_PREC = lax.Precision.HIGHEST


def _fc0(x, W, b2):
    def body(x_ref, w_ref, b_ref, o_ref):
        acc = jnp.dot(x_ref[...], w_ref[...], precision=_PREC,
                      preferred_element_type=jnp.float32)
        o_ref[...] = jnp.maximum(acc + b_ref[...], 0.0)[None]

    return pl.pallas_call(
        body,
        grid=(_NRB, 2),
        in_specs=[
            pl.BlockSpec((_RB, N_FEAT), lambda r, n: (r, 0)),
            pl.BlockSpec((N_FEAT, HALF), lambda r, n: (0, n)),
            pl.BlockSpec((1, HALF), lambda r, n: (0, n)),
        ],
        out_specs=pl.BlockSpec((1, _RB, HALF), lambda r, n: (n, r, 0)),
        out_shape=jax.ShapeDtypeStruct((2, N_NODE, HALF), jnp.float32),
    )(x, W, b2)


def _layer(hi2, h02, W3, theta):
    def body(hi_ref, h0_ref, w_ref, o_ref):
        n = pl.program_id(1)
        s0 = (1.0 - ALPHA) * hi_ref[0] + ALPHA * h0_ref[0]
        s1 = (1.0 - ALPHA) * hi_ref[1] + ALPHA * h0_ref[1]
        mm = (jnp.dot(s0, w_ref[0], precision=_PREC,
                      preferred_element_type=jnp.float32)
              + jnp.dot(s1, w_ref[1], precision=_PREC,
                        preferred_element_type=jnp.float32))
        sn = jnp.where(n == 0, s0, s1)
        o_ref[...] = jnp.maximum(theta * mm + (1.0 - theta) * sn, 0.0)[None]

    return pl.pallas_call(
        body,
        grid=(_NRB, 2),
        in_specs=[
            pl.BlockSpec((2, _RB, HALF), lambda r, n: (0, r, 0)),
            pl.BlockSpec((2, _RB, HALF), lambda r, n: (0, r, 0)),
            pl.BlockSpec((2, HALF, HALF), lambda r, n: (0, 0, n)),
        ],
        out_specs=pl.BlockSpec((1, _RB, HALF), lambda r, n: (n, r, 0)),
        out_shape=jax.ShapeDtypeStruct((2, N_NODE, HALF), jnp.float32),
    )(hi2, h02, W3)


def _head(h2, W3, b2):
    def body(h_ref, w_ref, b_ref, o_ref):
        l = (jnp.dot(h_ref[0], w_ref[0], precision=_PREC,
                     preferred_element_type=jnp.float32)
             + jnp.dot(h_ref[1], w_ref[1], precision=_PREC,
                       preferred_element_type=jnp.float32)
             + b_ref[...])
        m = jnp.max(l, axis=1, keepdims=True)
        e = jnp.exp(l - m)
        lse = jnp.log(jnp.sum(e, axis=1, keepdims=True)) + m
        o_ref[...] = l - lse

    return pl.pallas_call(
        body,
        grid=(_NRB,),
        in_specs=[
            pl.BlockSpec((2, _RB, HALF), lambda r: (0, r, 0)),
            pl.BlockSpec((2, HALF, N_CLASS), lambda r: (0, 0, 0)),
            pl.BlockSpec((1, N_CLASS), lambda r: (0, 0)),
        ],
        out_specs=pl.BlockSpec((_RB, N_CLASS), lambda r: (r, 0)),
        out_shape=jax.ShapeDtypeStruct((N_NODE, N_CLASS), jnp.float32),
    )(h2, W3, b2)


def _prep_edges(indices, values):
    dst = indices[0].astype(jnp.int32)
    col = indices[1].astype(jnp.int32)
    pad = EPAD - N_EDGE
    dst = jnp.concatenate([dst, jnp.zeros((pad,), jnp.int32)])
    col = jnp.concatenate([col, jnp.zeros((pad,), jnp.int32)])
    val = jnp.concatenate([values.astype(jnp.float32),
                           jnp.zeros((pad,), jnp.float32)])
    return col, dst, val


def kernel(x, support0_indices, support0_values, support1_indices,
           support1_values, fc0_W, fc0_b, fc1_W, fc1_b, conv_kernels,
           conv_weights):
    col0, dst0, val0 = _prep_edges(support0_indices, support0_values)
    col1, dst1, val1 = _prep_edges(support1_indices, support1_values)

    h2 = _fc0(x, fc0_W, fc0_b.reshape(1, N_FEAT))
    h02 = h2
    cw3 = conv_weights.reshape(N_LAYERS, 2, HALF, N_HID)

    for i in range(N_LAYERS):
        theta = math.log(LAMDA / (i + 1) + 1.0)
        t = _spmm(h2.reshape(2 * N_NODE, HALF), col1, dst1, val1)
        t = _scale_rows(t.reshape(2, N_NODE, HALF), conv_kernels[i].reshape(N_NODE, 1))
        hi = _spmm(t.reshape(2 * N_NODE, HALF), col0, dst0, val0)
        h2 = _layer(hi.reshape(2, N_NODE, HALF), h02, cw3[i], theta)

    return _head(h2, fc1_W.reshape(2, HALF, N_CLASS),
                 fc1_b.reshape(1, N_CLASS))


# speedup vs baseline: 3.1522x; 1.5916x over previous
import functools
import math

import jax
import jax.numpy as jnp
from jax import lax
from jax.experimental import pallas as pl
from jax.experimental.pallas import tpu as pltpu
from jax.experimental.pallas import tpu_sc as plsc

N_NODE = 10000
N_FEAT = 256
N_HID = 256
N_CLASS = 64
N_LAYERS = 8
N_EDGE = 160000
LAMDA = 0.5
ALPHA = 0.1

HALF = 128
CHUNK = 128
EPAD = 163840
E_PER_TILE = EPAD // 16
N_CHUNKS = E_PER_TILE // CHUNK
ROWS_PER_TILE = 624
ROW_CHUNK = 208


def _make_spmm():
    mesh = plsc.VectorSubcoreMesh(core_axis_name="c", subcore_axis_name="s")

    scratch = [
        pltpu.VMEM((N_CHUNKS, CHUNK), jnp.int32),
        pltpu.VMEM((2, CHUNK), jnp.int32),
        pltpu.VMEM((2, CHUNK), jnp.float32),
        pltpu.VMEM((2 * CHUNK, HALF), jnp.float32),
        pltpu.VMEM_SHARED((N_NODE, HALF), jnp.float32),
        pltpu.SemaphoreType.DMA((2,)),
        pltpu.SemaphoreType.DMA((2,)),
        pltpu.SemaphoreType.DMA((2,)),
    ]

    def body(table, col, dst, val, out,
             colbuf, dstb, wb, rows, acc, gsem, ssem, csem):
        c = lax.axis_index("c")
        s = lax.axis_index("s")

        pltpu.sync_copy(col.at[c * 16 + s], colbuf)

        def zrow(r, carry):
            for f in range(HALF // 16):
                rows[r, pl.ds(f * 16, 16)] = jnp.zeros((16,), jnp.float32)
            return carry
        lax.fori_loop(0, ROW_CHUNK, zrow, 0)
        for j in range(3):
            pltpu.sync_copy(
                rows.at[pl.ds(0, ROW_CHUNK)],
                acc.at[pl.ds(s * ROWS_PER_TILE + j * ROW_CHUNK, ROW_CHUNK)])

        @pl.when(s == 0)
        def _():
            pltpu.sync_copy(rows.at[pl.ds(0, 16)], acc.at[pl.ds(9984, 16)])

        plsc.subcore_barrier()

        dvbase = s * N_CHUNKS
        pltpu.async_copy(dst.at[dvbase], dstb.at[0], csem.at[0])
        pltpu.async_copy(val.at[dvbase], wb.at[0], csem.at[0])
        pltpu.async_copy(table.at[colbuf.at[0]], rows.at[pl.ds(0, CHUNK)],
                         gsem.at[0])

        def chunk(g, carry):
            b = lax.rem(g, 2)
            base_r = b * CHUNK
            obase = CHUNK - base_r
            pltpu.make_async_copy(table.at[colbuf.at[g]],
                                  rows.at[pl.ds(base_r, CHUNK)],
                                  gsem.at[b]).wait()
            @pl.when(g >= 1)
            def _():
                pltpu.make_async_copy(rows.at[pl.ds(obase, CHUNK)],
                                      acc.at[dstb.at[1 - b]],
                                      ssem.at[1 - b]).wait()

            @pl.when(g + 1 < N_CHUNKS)
            def _():
                pltpu.async_copy(dst.at[dvbase + g + 1], dstb.at[1 - b],
                                 csem.at[1 - b])
                pltpu.async_copy(val.at[dvbase + g + 1], wb.at[1 - b],
                                 csem.at[1 - b])
                pltpu.async_copy(table.at[colbuf.at[g + 1]],
                                 rows.at[pl.ds(obase, CHUNK)],
                                 gsem.at[1 - b])
            pltpu.make_async_copy(dst.at[dvbase + g], dstb.at[b],
                                  csem.at[b]).wait()
            pltpu.make_async_copy(val.at[dvbase + g], wb.at[b],
                                  csem.at[b]).wait()
            for eg in range(CHUNK // 16):
                w16 = wb[b, pl.ds(eg * 16, 16)]
                for e in range(16):
                    r = eg * 16 + e
                    sp = jnp.broadcast_to(w16[e], (16,))
                    for f in range(HALF // 16):
                        rows[base_r + r, pl.ds(f * 16, 16)] = (
                            rows[base_r + r, pl.ds(f * 16, 16)] * sp)
            pltpu.async_copy(rows.at[pl.ds(base_r, CHUNK)],
                             acc.at[dstb.at[b]], ssem.at[b], add=True)
            return carry

        lax.fori_loop(0, N_CHUNKS, chunk, 0)

        bb = (N_CHUNKS - 1) % 2
        pltpu.make_async_copy(rows.at[pl.ds(bb * CHUNK, CHUNK)],
                              acc.at[dstb.at[bb]], ssem.at[bb]).wait()

        plsc.subcore_barrier()

        coff = c * N_NODE
        for j in range(3):
            r0 = s * ROWS_PER_TILE + j * ROW_CHUNK
            pltpu.sync_copy(acc.at[pl.ds(r0, ROW_CHUNK)],
                            out.at[pl.ds(coff + r0, ROW_CHUNK)])

        @pl.when(s == 0)
        def _():
            pltpu.sync_copy(acc.at[pl.ds(9984, 16)],
                            out.at[pl.ds(coff + 9984, 16)])

    def run(*args):
        return pl.kernel(
            body,
            out_type=jax.ShapeDtypeStruct((2 * N_NODE, HALF), jnp.float32),
            mesh=mesh,
            scratch_types=scratch,
        )(*args)

    return run


_spmm = _make_spmm()


def _scale_rows(t2, k):
    def body(t_ref, k_ref, o_ref):
        o_ref[...] = t_ref[...] * k_ref[...][None, :, :]

    return pl.pallas_call(
        body,
        grid=(_NRB,),
        in_specs=[
            pl.BlockSpec((2, _RB, HALF), lambda r: (0, r, 0)),
            pl.BlockSpec((_RB, 1), lambda r: (r, 0)),
        ],
        out_specs=pl.BlockSpec((2, _RB, HALF), lambda r: (0, r, 0)),
        out_shape=jax.ShapeDtypeStruct((2, N_NODE, HALF), jnp.float32),
    )(t2, k)


_RB = 1000
_NRB = N_NODE // _RB
_PREC = lax.Precision.HIGHEST


def _fc0(x, W, b2):
    def body(x_ref, w_ref, b_ref, o_ref):
        acc = jnp.dot(x_ref[...], w_ref[...], precision=_PREC,
                      preferred_element_type=jnp.float32)
        o_ref[...] = jnp.maximum(acc + b_ref[...], 0.0)[None]

    return pl.pallas_call(
        body,
        grid=(_NRB, 2),
        in_specs=[
            pl.BlockSpec((_RB, N_FEAT), lambda r, n: (r, 0)),
            pl.BlockSpec((N_FEAT, HALF), lambda r, n: (0, n)),
            pl.BlockSpec((1, HALF), lambda r, n: (0, n)),
        ],
        out_specs=pl.BlockSpec((1, _RB, HALF), lambda r, n: (n, r, 0)),
        out_shape=jax.ShapeDtypeStruct((2, N_NODE, HALF), jnp.float32),
    )(x, W, b2)


def _layer(hi2, h02, W3, theta):
    def body(hi_ref, h0_ref, w_ref, o_ref):
        n = pl.program_id(1)
        s0 = (1.0 - ALPHA) * hi_ref[0] + ALPHA * h0_ref[0]
        s1 = (1.0 - ALPHA) * hi_ref[1] + ALPHA * h0_ref[1]
        mm = (jnp.dot(s0, w_ref[0], precision=_PREC,
                      preferred_element_type=jnp.float32)
              + jnp.dot(s1, w_ref[1], precision=_PREC,
                        preferred_element_type=jnp.float32))
        sn = jnp.where(n == 0, s0, s1)
        o_ref[...] = jnp.maximum(theta * mm + (1.0 - theta) * sn, 0.0)[None]

    return pl.pallas_call(
        body,
        grid=(_NRB, 2),
        in_specs=[
            pl.BlockSpec((2, _RB, HALF), lambda r, n: (0, r, 0)),
            pl.BlockSpec((2, _RB, HALF), lambda r, n: (0, r, 0)),
            pl.BlockSpec((2, HALF, HALF), lambda r, n: (0, 0, n)),
        ],
        out_specs=pl.BlockSpec((1, _RB, HALF), lambda r, n: (n, r, 0)),
        out_shape=jax.ShapeDtypeStruct((2, N_NODE, HALF), jnp.float32),
    )(hi2, h02, W3)


def _head(h2, W3, b2):
    def body(h_ref, w_ref, b_ref, o_ref):
        l = (jnp.dot(h_ref[0], w_ref[0], precision=_PREC,
                     preferred_element_type=jnp.float32)
             + jnp.dot(h_ref[1], w_ref[1], precision=_PREC,
                       preferred_element_type=jnp.float32)
             + b_ref[...])
        m = jnp.max(l, axis=1, keepdims=True)
        e = jnp.exp(l - m)
        lse = jnp.log(jnp.sum(e, axis=1, keepdims=True)) + m
        o_ref[...] = l - lse

    return pl.pallas_call(
        body,
        grid=(_NRB,),
        in_specs=[
            pl.BlockSpec((2, _RB, HALF), lambda r: (0, r, 0)),
            pl.BlockSpec((2, HALF, N_CLASS), lambda r: (0, 0, 0)),
            pl.BlockSpec((1, N_CLASS), lambda r: (0, 0)),
        ],
        out_specs=pl.BlockSpec((_RB, N_CLASS), lambda r: (r, 0)),
        out_shape=jax.ShapeDtypeStruct((N_NODE, N_CLASS), jnp.float32),
    )(h2, W3, b2)


def _prep_edges(indices, values):
    dst = indices[0].astype(jnp.int32)
    col = indices[1].astype(jnp.int32)
    pad = EPAD - N_EDGE
    dst = jnp.concatenate([dst, jnp.zeros((pad,), jnp.int32)])
    col = jnp.concatenate([col, jnp.zeros((pad,), jnp.int32)])
    val = jnp.concatenate([values.astype(jnp.float32),
                           jnp.zeros((pad,), jnp.float32)])
    col2 = jnp.stack([col, col + N_NODE]).reshape(32, N_CHUNKS, CHUNK)
    return (col2, dst.reshape(16 * N_CHUNKS, CHUNK),
            val.reshape(16 * N_CHUNKS, CHUNK))


def kernel(x, support0_indices, support0_values, support1_indices,
           support1_values, fc0_W, fc0_b, fc1_W, fc1_b, conv_kernels,
           conv_weights):
    col0, dst0, val0 = _prep_edges(support0_indices, support0_values)
    col1, dst1, val1 = _prep_edges(support1_indices, support1_values)

    h2 = _fc0(x, fc0_W, fc0_b.reshape(1, N_FEAT))
    h02 = h2
    cw3 = conv_weights.reshape(N_LAYERS, 2, HALF, N_HID)

    for i in range(N_LAYERS):
        theta = math.log(LAMDA / (i + 1) + 1.0)
        t = _spmm(h2.reshape(2 * N_NODE, HALF), col1, dst1, val1)
        t = _scale_rows(t.reshape(2, N_NODE, HALF), conv_kernels[i].reshape(N_NODE, 1))
        hi = _spmm(t.reshape(2 * N_NODE, HALF), col0, dst0, val0)
        h2 = _layer(hi.reshape(2, N_NODE, HALF), h02, cw3[i], theta)

    return _head(h2, fc1_W.reshape(2, HALF, N_CLASS),
                 fc1_b.reshape(1, N_CLASS))
